# Initial kernel scaffold; baseline (speedup 1.0000x reference)
import jax
import jax.numpy as jnp
from jax.experimental import pallas as pl


def kernel(x, edge_index, batch, W1, b1, W2, b2, W3, b3, Wp, bp):
    raise NotImplementedError("write your pallas kernel here")


# speedup vs baseline: 10.4110x; 10.4110x over previous
import functools

import jax
import jax.numpy as jnp
from jax import lax
from jax.experimental import pallas as pl
from jax.experimental.pallas import tpu as pltpu
from jax.experimental.pallas import tpu_sc as plsc

N = 10000
NPAD = 10240
E = 160000
EPAD = 163840
F_IN = 256
H = 512
G = 64

NC = 2
NS = 16
CK = 128
NCH = EPAD // (NS * CK)
EPT32 = EPAD // 32
NG32 = EPT32 // 16

BN = 512
GT = NPAD // BN
RZ = NPAD // NS

_mesh = plsc.VectorSubcoreMesh(core_axis_name="c", subcore_axis_name="s")
_sc_params = pltpu.CompilerParams(needs_layout_passes=False)


@functools.partial(
    pl.kernel,
    out_type=jax.ShapeDtypeStruct((32, NPAD), jnp.float32),
    mesh=_mesh,
    compiler_params=_sc_params,
    scratch_types=[
        pltpu.VMEM((NG32, 16), jnp.int32),
        pltpu.VMEM((NPAD,), jnp.float32),
    ],
)
def _deg_kernel(dst_hbm, out_hbm, dstb, hist):
  wid = lax.axis_index("s") * NC + lax.axis_index("c")
  zeros16 = jnp.zeros((16,), jnp.float32)
  ones16 = jnp.ones((16,), jnp.float32)

  def zero_body(i, _):
    hist[pl.ds(i * 16, 16)] = zeros16
    return 0

  lax.fori_loop(0, NPAD // 16, zero_body, 0)
  pltpu.sync_copy(dst_hbm.at[wid], dstb)

  def edge_body(j, _):
    idx = dstb[j]
    plsc.addupdate_scatter(hist, [idx], ones16)
    return 0

  lax.fori_loop(0, NG32, edge_body, 0)
  pltpu.sync_copy(hist, out_hbm.at[wid])


def _make_spmm(nb):
  nbps = nb // 2

  @functools.partial(
      pl.kernel,
      out_type=jax.ShapeDtypeStruct((nb * NPAD, 128), jnp.float32),
      mesh=_mesh,
      compiler_params=_sc_params,
      scratch_types=[
          pltpu.VMEM((CK,), jnp.int32),
          pltpu.VMEM((CK,), jnp.int32),
          pltpu.VMEM((CK,), jnp.int32),
          pltpu.VMEM((CK,), jnp.int32),
          pltpu.VMEM((CK, 128), jnp.float32),
          pltpu.VMEM((CK, 128), jnp.float32),
          pltpu.VMEM_SHARED((NPAD, 128), jnp.float32),
          pltpu.SemaphoreType.DMA,
          pltpu.SemaphoreType.DMA,
      ],
  )
  def spmm(src_hbm, dst_hbm, h_hbm, zeros_hbm, out_hbm,
           sidx0, sidx1, didx0, didx1, rows0, rows1, acc, sem0, sem1):
    cid = lax.axis_index("c")
    sid = lax.axis_index("s")
    for bi in range(nbps):
      b = cid * nbps + bi
      pltpu.sync_copy(zeros_hbm, acc.at[pl.ds(sid * RZ, RZ)])
      plsc.subcore_barrier()

      def chunk_pair(j, _):
        k0 = 2 * j
        k1 = k0 + 1
        pltpu.sync_copy(src_hbm.at[b, sid, k0], sidx0)
        pltpu.sync_copy(dst_hbm.at[sid, k0], didx0)
        g0 = pltpu.async_copy(h_hbm.at[sidx0], rows0, sem0)
        pltpu.sync_copy(src_hbm.at[b, sid, k1], sidx1)
        pltpu.sync_copy(dst_hbm.at[sid, k1], didx1)
        g1 = pltpu.async_copy(h_hbm.at[sidx1], rows1, sem1)
        g0.wait()
        pltpu.sync_copy(rows0, acc.at[didx0], add=True)
        g1.wait()
        pltpu.sync_copy(rows1, acc.at[didx1], add=True)
        return 0

      lax.fori_loop(0, NCH // 2, chunk_pair, 0)
      plsc.subcore_barrier()
      pltpu.sync_copy(acc.at[pl.ds(sid * RZ, RZ)],
                      out_hbm.at[pl.ds(b * NPAD + sid * RZ, RZ)])
      plsc.subcore_barrier()

  return spmm


_spmm2 = _make_spmm(2)
_spmm4 = _make_spmm(4)


@functools.partial(
    pl.kernel,
    out_type=jax.ShapeDtypeStruct((32, NPAD), jnp.float32),
    mesh=_mesh,
    compiler_params=_sc_params,
    scratch_types=[
        pltpu.VMEM((NG32, 16), jnp.int32),
        pltpu.VMEM((NG32, 16), jnp.int32),
        pltpu.VMEM((NPAD,), jnp.float32),
        pltpu.VMEM((NPAD,), jnp.float32),
    ],
)
def _spmv_kernel(src_hbm, dst_hbm, v_hbm, out_hbm, srcb, dstb, vv, hist):
  wid = lax.axis_index("s") * NC + lax.axis_index("c")
  zeros16 = jnp.zeros((16,), jnp.float32)

  def zero_body(i, _):
    hist[pl.ds(i * 16, 16)] = zeros16
    return 0

  lax.fori_loop(0, NPAD // 16, zero_body, 0)
  pltpu.sync_copy(v_hbm, vv)
  pltpu.sync_copy(src_hbm.at[wid], srcb)
  pltpu.sync_copy(dst_hbm.at[wid], dstb)

  def edge_body(j, _):
    s16 = srcb[j]
    d16 = dstb[j]
    vals = plsc.load_gather(vv, [s16])
    plsc.addupdate_scatter(hist, [d16], vals)
    return 0

  lax.fori_loop(0, NG32, edge_body, 0)
  pltpu.sync_copy(hist, out_hbm.at[wid])


def _prep_body(degp_ref, x_ref, dinv_ref, xb_ref):
  ones32 = jnp.ones((32, 1), jnp.float32)
  deg = lax.dot_general(degp_ref[...], ones32, (((0,), (0,)), ((), ())),
                        preferred_element_type=jnp.float32)
  dinv = lax.rsqrt(deg + 1.0)
  dinv_ref[...] = dinv
  x = x_ref[...]
  for b in range(2):
    xb_ref[b] = dinv * x[:, b * 128:(b + 1) * 128]


def _prep_call(degp, xpad):
  return pl.pallas_call(
      _prep_body,
      grid=(GT,),
      in_specs=[
          pl.BlockSpec((32, BN), lambda i: (0, i)),
          pl.BlockSpec((BN, F_IN), lambda i: (i, 0)),
      ],
      out_specs=[
          pl.BlockSpec((BN, 1), lambda i: (i, 0)),
          pl.BlockSpec((2, BN, 128), lambda i: (0, i, 0)),
      ],
      out_shape=[
          jax.ShapeDtypeStruct((NPAD, 1), jnp.float32),
          jax.ShapeDtypeStruct((2, NPAD, 128), jnp.float32),
      ],
  )(degp, xpad)


def _layer1_body(accx_ref, xb_ref, dinv_ref, w1_ref, b1_ref, h1b_ref):
  dinv = dinv_ref[...]
  sx = jnp.concatenate(
      [dinv * (accx_ref[b] + xb_ref[b]) for b in range(2)], axis=1)
  h = jnp.dot(sx, w1_ref[...], preferred_element_type=jnp.float32)
  h = jnp.maximum(h + b1_ref[...], 0.0)
  for b in range(4):
    h1b_ref[b] = dinv * h[:, b * 128:(b + 1) * 128]


def _layer1_call(accx, xb, dinv, W1, b1r):
  return pl.pallas_call(
      _layer1_body,
      grid=(GT,),
      in_specs=[
          pl.BlockSpec((2, BN, 128), lambda i: (0, i, 0)),
          pl.BlockSpec((2, BN, 128), lambda i: (0, i, 0)),
          pl.BlockSpec((BN, 1), lambda i: (i, 0)),
          pl.BlockSpec((F_IN, H), lambda i: (0, 0)),
          pl.BlockSpec((1, H), lambda i: (0, 0)),
      ],
      out_specs=pl.BlockSpec((4, BN, 128), lambda i: (0, i, 0)),
      out_shape=jax.ShapeDtypeStruct((4, NPAD, 128), jnp.float32),
  )(accx, xb, dinv, W1, b1r)


def _layer2_body(acch_ref, h1b_ref, dinv_ref, w2_ref, b2_ref, u_ref, vp_ref):
  dinv = dinv_ref[...]
  sh = jnp.concatenate(
      [dinv * (acch_ref[b] + h1b_ref[b]) for b in range(4)], axis=1)
  h2 = jnp.dot(sh, w2_ref[...], preferred_element_type=jnp.float32)
  h2 = jnp.maximum(h2 + b2_ref[...], 0.0)
  v = jnp.dot(h2, u_ref[...], preferred_element_type=jnp.float32)
  vp_ref[...] = dinv * v


def _layer2_call(acch, h1b, dinv, W2, b2r, u):
  return pl.pallas_call(
      _layer2_body,
      grid=(GT,),
      in_specs=[
          pl.BlockSpec((4, BN, 128), lambda i: (0, i, 0)),
          pl.BlockSpec((4, BN, 128), lambda i: (0, i, 0)),
          pl.BlockSpec((BN, 1), lambda i: (i, 0)),
          pl.BlockSpec((H, H), lambda i: (0, 0)),
          pl.BlockSpec((1, H), lambda i: (0, 0)),
          pl.BlockSpec((H, 1), lambda i: (0, 0)),
      ],
      out_specs=pl.BlockSpec((BN, 1), lambda i: (i, 0)),
      out_shape=jax.ShapeDtypeStruct((NPAD, 1), jnp.float32),
  )(acch, h1b, dinv, W2, b2r, u)


def _u_body(w3_ref, wp_ref, u_ref):
  u_ref[...] = jnp.dot(w3_ref[...], wp_ref[...],
                       preferred_element_type=jnp.float32)


def _u_call(W3, Wp):
  return pl.pallas_call(
      _u_body,
      out_shape=jax.ShapeDtypeStruct((H, 1), jnp.float32),
  )(W3, Wp)


def _pool_body(accv_ref, vp_ref, dinv_ref, batch_ref, b3_ref, wp_ref, bp_ref,
               out_ref):
  accsum = jnp.sum(accv_ref[...], axis=0, keepdims=True)
  w = dinv_ref[...] * (accsum + vp_ref[...])
  ids = lax.broadcasted_iota(jnp.int32, (G, NPAD), 0)
  m = (batch_ref[...] == ids).astype(jnp.float32)
  counts = jnp.sum(m, axis=1, keepdims=True)
  sums = jnp.sum(m * w, axis=1, keepdims=True)
  c3 = jnp.sum(b3_ref[...] * wp_ref[...])
  pooled = sums / jnp.maximum(counts, 1.0)
  out_ref[...] = pooled + jnp.where(counts > 0.0, c3, 0.0) + bp_ref[...]


def _pool_call(accv, vp_row, dinv_row, batch_row, b3r, wpr, bpr):
  return pl.pallas_call(
      _pool_body,
      out_shape=jax.ShapeDtypeStruct((G, 1), jnp.float32),
  )(accv, vp_row, dinv_row, batch_row, b3r, wpr, bpr)


@jax.jit
def kernel(x, edge_index, batch, W1, b1, W2, b2, W3, b3, Wp, bp):
  src = edge_index[0]
  dst = edge_index[1]
  padn = EPAD - E
  srcp = jnp.concatenate([src, jnp.zeros((padn,), jnp.int32)])
  dstp = jnp.concatenate([dst, jnp.full((padn,), N, jnp.int32)])

  dst16 = dstp.reshape(NS, NCH, CK)
  src32 = srcp.reshape(32, NG32, 16)
  dst32 = dstp.reshape(32, NG32, 16)
  off2 = (srcp[None, :] + (jnp.arange(2, dtype=jnp.int32) * NPAD)[:, None])
  off4 = (srcp[None, :] + (jnp.arange(4, dtype=jnp.int32) * NPAD)[:, None])
  src2 = off2.reshape(2, NS, NCH, CK)
  src4 = off4.reshape(4, NS, NCH, CK)

  xpad = jnp.concatenate(
      [x.astype(jnp.float32), jnp.zeros((NPAD - N, F_IN), jnp.float32)])
  batchp = jnp.concatenate(
      [batch, jnp.full((NPAD - N,), G, jnp.int32)]).reshape(1, NPAD)
  zeros_blk = jnp.zeros((RZ, 128), jnp.float32)

  degp = _deg_kernel(dst32)
  dinv, xb = _prep_call(degp, xpad)
  accx = _spmm2(src2, dst16, xb.reshape(2 * NPAD, 128), zeros_blk)
  h1b = _layer1_call(accx.reshape(2, NPAD, 128), xb, dinv, W1,
                     b1.reshape(1, H))
  acch = _spmm4(src4, dst16, h1b.reshape(4 * NPAD, 128), zeros_blk)
  u = _u_call(W3, Wp)
  vp = _layer2_call(acch.reshape(4, NPAD, 128), h1b, dinv, W2,
                    b2.reshape(1, H), u)
  accv = _spmv_kernel(src32, dst32, vp.reshape(NPAD))
  return _pool_call(accv, vp.reshape(1, NPAD), dinv.reshape(1, NPAD),
                    batchp, b3.reshape(1, H), Wp.reshape(1, H),
                    bp.reshape(1, 1))


# speedup vs baseline: 11.2879x; 1.0842x over previous
import functools

import jax
import jax.numpy as jnp
from jax import lax
from jax.experimental import pallas as pl
from jax.experimental.pallas import tpu as pltpu
from jax.experimental.pallas import tpu_sc as plsc

N = 10000
NPAD = 10240
E = 160000
EPAD = 163840
F_IN = 256
H = 512
G = 64

NC = 2
NS = 16
CK = 64
NCH = EPAD // (NS * CK)
NSC = NCH // 4
EPT32 = EPAD // 32
NG32 = EPT32 // 16

BN = 512
GT = NPAD // BN
RZ = NPAD // NS

_mesh = plsc.VectorSubcoreMesh(core_axis_name="c", subcore_axis_name="s")
_sc_params = pltpu.CompilerParams(needs_layout_passes=False)


@functools.partial(
    pl.kernel,
    out_type=jax.ShapeDtypeStruct((32, NPAD), jnp.float32),
    mesh=_mesh,
    compiler_params=_sc_params,
    scratch_types=[
        pltpu.VMEM((NG32, 16), jnp.int32),
        pltpu.VMEM((NPAD,), jnp.float32),
    ],
)
def _deg_kernel(dst_hbm, out_hbm, dstb, hist):
  wid = lax.axis_index("s") * NC + lax.axis_index("c")
  zeros16 = jnp.zeros((16,), jnp.float32)
  ones16 = jnp.ones((16,), jnp.float32)

  def zero_body(i, _):
    hist[pl.ds(i * 16, 16)] = zeros16
    return 0

  lax.fori_loop(0, NPAD // 16, zero_body, 0)
  pltpu.sync_copy(dst_hbm.at[wid], dstb)

  def edge_body(j, _):
    idx = dstb[j]
    plsc.addupdate_scatter(hist, [idx], ones16)
    return 0

  lax.fori_loop(0, NG32, edge_body, 0)
  pltpu.sync_copy(hist, out_hbm.at[wid])


def _make_spmm(nb):
  nbps = nb // 2

  @functools.partial(
      pl.kernel,
      out_type=jax.ShapeDtypeStruct((nb * NPAD, 128), jnp.float32),
      mesh=_mesh,
      compiler_params=_sc_params,
      scratch_types=[
          pltpu.VMEM((2, 4, CK), jnp.int32),
          pltpu.VMEM((2, 4, CK), jnp.int32),
          pltpu.VMEM((CK, 128), jnp.float32),
          pltpu.VMEM((CK, 128), jnp.float32),
          pltpu.VMEM((CK, 128), jnp.float32),
          pltpu.VMEM((CK, 128), jnp.float32),
          pltpu.VMEM_SHARED((NPAD, 128), jnp.float32),
          pltpu.SemaphoreType.DMA,
          pltpu.SemaphoreType.DMA,
          pltpu.SemaphoreType.DMA,
          pltpu.SemaphoreType.DMA,
      ],
  )
  def spmm(src_hbm, dst_hbm, h_hbm, zeros_hbm, out_hbm,
           sidx, didx, r0, r1, r2, r3, acc, m0, m1, m2, m3):
    cid = lax.axis_index("c")
    sid = lax.axis_index("s")
    rows = (r0, r1, r2, r3)
    sems = (m0, m1, m2, m3)

    def gather(p, c, buf):
      return pltpu.async_copy(h_hbm.at[sidx.at[p, c]], rows[buf], sems[buf])

    def gwait(p, c, buf):
      pltpu.make_async_copy(h_hbm.at[sidx.at[p, c]], rows[buf],
                            sems[buf]).wait()

    def scatter(p, c, buf):
      pltpu.sync_copy(rows[buf], acc.at[didx.at[p, c]], add=True)

    for bi in range(nbps):
      b = cid * nbps + bi
      pltpu.sync_copy(zeros_hbm, acc.at[pl.ds(sid * RZ, RZ)])
      plsc.subcore_barrier()

      pltpu.sync_copy(src_hbm.at[b, sid, pl.ds(0, 4)], sidx.at[0])
      pltpu.sync_copy(dst_hbm.at[sid, pl.ds(0, 4)], didx.at[0])
      gather(0, 0, 0)
      gather(0, 1, 1)

      def superchunk(j, _):
        p = lax.rem(j, 2)
        pn = lax.rem(j + 1, 2)
        not_last = j < NSC - 1

        @pl.when(not_last)
        def _():
          pltpu.sync_copy(src_hbm.at[b, sid, pl.ds(4 * (j + 1), 4)],
                          sidx.at[pn])
          pltpu.sync_copy(dst_hbm.at[sid, pl.ds(4 * (j + 1), 4)],
                          didx.at[pn])

        gwait(p, 0, 0)
        gather(p, 2, 2)
        scatter(p, 0, 0)
        gwait(p, 1, 1)
        gather(p, 3, 3)
        scatter(p, 1, 1)
        gwait(p, 2, 2)

        @pl.when(not_last)
        def _():
          gather(pn, 0, 0)

        scatter(p, 2, 2)
        gwait(p, 3, 3)

        @pl.when(not_last)
        def _():
          gather(pn, 1, 1)

        scatter(p, 3, 3)
        return 0

      lax.fori_loop(0, NSC, superchunk, 0)
      plsc.subcore_barrier()
      pltpu.sync_copy(acc.at[pl.ds(sid * RZ, RZ)],
                      out_hbm.at[pl.ds(b * NPAD + sid * RZ, RZ)])
      plsc.subcore_barrier()

  return spmm


_spmm2 = _make_spmm(2)
_spmm4 = _make_spmm(4)


@functools.partial(
    pl.kernel,
    out_type=jax.ShapeDtypeStruct((32, NPAD), jnp.float32),
    mesh=_mesh,
    compiler_params=_sc_params,
    scratch_types=[
        pltpu.VMEM((NG32, 16), jnp.int32),
        pltpu.VMEM((NG32, 16), jnp.int32),
        pltpu.VMEM((NPAD,), jnp.float32),
        pltpu.VMEM((NPAD,), jnp.float32),
    ],
)
def _spmv_kernel(src_hbm, dst_hbm, v_hbm, out_hbm, srcb, dstb, vv, hist):
  wid = lax.axis_index("s") * NC + lax.axis_index("c")
  zeros16 = jnp.zeros((16,), jnp.float32)

  def zero_body(i, _):
    hist[pl.ds(i * 16, 16)] = zeros16
    return 0

  lax.fori_loop(0, NPAD // 16, zero_body, 0)
  pltpu.sync_copy(v_hbm, vv)
  pltpu.sync_copy(src_hbm.at[wid], srcb)
  pltpu.sync_copy(dst_hbm.at[wid], dstb)

  def edge_body(j, _):
    s16 = srcb[j]
    d16 = dstb[j]
    vals = plsc.load_gather(vv, [s16])
    plsc.addupdate_scatter(hist, [d16], vals)
    return 0

  lax.fori_loop(0, NG32, edge_body, 0)
  pltpu.sync_copy(hist, out_hbm.at[wid])


def _prep_body(degp_ref, x_ref, dinv_ref, xb_ref):
  ones32 = jnp.ones((32, 1), jnp.float32)
  deg = lax.dot_general(degp_ref[...], ones32, (((0,), (0,)), ((), ())),
                        preferred_element_type=jnp.float32)
  dinv = lax.rsqrt(deg + 1.0)
  dinv_ref[...] = dinv
  x = x_ref[...]
  for b in range(2):
    xb_ref[b] = dinv * x[:, b * 128:(b + 1) * 128]


def _prep_call(degp, xpad):
  return pl.pallas_call(
      _prep_body,
      grid=(GT,),
      in_specs=[
          pl.BlockSpec((32, BN), lambda i: (0, i)),
          pl.BlockSpec((BN, F_IN), lambda i: (i, 0)),
      ],
      out_specs=[
          pl.BlockSpec((BN, 1), lambda i: (i, 0)),
          pl.BlockSpec((2, BN, 128), lambda i: (0, i, 0)),
      ],
      out_shape=[
          jax.ShapeDtypeStruct((NPAD, 1), jnp.float32),
          jax.ShapeDtypeStruct((2, NPAD, 128), jnp.float32),
      ],
  )(degp, xpad)


def _layer1_body(accx_ref, xb_ref, dinv_ref, w1_ref, b1_ref, h1b_ref):
  dinv = dinv_ref[...]
  sx = jnp.concatenate(
      [dinv * (accx_ref[b] + xb_ref[b]) for b in range(2)], axis=1)
  h = jnp.dot(sx, w1_ref[...], preferred_element_type=jnp.float32)
  h = jnp.maximum(h + b1_ref[...], 0.0)
  for b in range(4):
    h1b_ref[b] = dinv * h[:, b * 128:(b + 1) * 128]


def _layer1_call(accx, xb, dinv, W1, b1r):
  return pl.pallas_call(
      _layer1_body,
      grid=(GT,),
      in_specs=[
          pl.BlockSpec((2, BN, 128), lambda i: (0, i, 0)),
          pl.BlockSpec((2, BN, 128), lambda i: (0, i, 0)),
          pl.BlockSpec((BN, 1), lambda i: (i, 0)),
          pl.BlockSpec((F_IN, H), lambda i: (0, 0)),
          pl.BlockSpec((1, H), lambda i: (0, 0)),
      ],
      out_specs=pl.BlockSpec((4, BN, 128), lambda i: (0, i, 0)),
      out_shape=jax.ShapeDtypeStruct((4, NPAD, 128), jnp.float32),
  )(accx, xb, dinv, W1, b1r)


def _layer2_body(acch_ref, h1b_ref, dinv_ref, w2_ref, b2_ref, u_ref, vp_ref):
  dinv = dinv_ref[...]
  sh = jnp.concatenate(
      [dinv * (acch_ref[b] + h1b_ref[b]) for b in range(4)], axis=1)
  h2 = jnp.dot(sh, w2_ref[...], preferred_element_type=jnp.float32)
  h2 = jnp.maximum(h2 + b2_ref[...], 0.0)
  v = jnp.dot(h2, u_ref[...], preferred_element_type=jnp.float32)
  vp_ref[...] = dinv * v


def _layer2_call(acch, h1b, dinv, W2, b2r, u):
  return pl.pallas_call(
      _layer2_body,
      grid=(GT,),
      in_specs=[
          pl.BlockSpec((4, BN, 128), lambda i: (0, i, 0)),
          pl.BlockSpec((4, BN, 128), lambda i: (0, i, 0)),
          pl.BlockSpec((BN, 1), lambda i: (i, 0)),
          pl.BlockSpec((H, H), lambda i: (0, 0)),
          pl.BlockSpec((1, H), lambda i: (0, 0)),
          pl.BlockSpec((H, 1), lambda i: (0, 0)),
      ],
      out_specs=pl.BlockSpec((BN, 1), lambda i: (i, 0)),
      out_shape=jax.ShapeDtypeStruct((NPAD, 1), jnp.float32),
  )(acch, h1b, dinv, W2, b2r, u)


def _u_body(w3_ref, wp_ref, u_ref):
  u_ref[...] = jnp.dot(w3_ref[...], wp_ref[...],
                       preferred_element_type=jnp.float32)


def _u_call(W3, Wp):
  return pl.pallas_call(
      _u_body,
      out_shape=jax.ShapeDtypeStruct((H, 1), jnp.float32),
  )(W3, Wp)


def _pool_body(accv_ref, vp_ref, dinv_ref, batch_ref, b3_ref, wp_ref, bp_ref,
               out_ref):
  accsum = jnp.sum(accv_ref[...], axis=0, keepdims=True)
  w = dinv_ref[...] * (accsum + vp_ref[...])
  ids = lax.broadcasted_iota(jnp.int32, (G, NPAD), 0)
  m = (batch_ref[...] == ids).astype(jnp.float32)
  counts = jnp.sum(m, axis=1, keepdims=True)
  sums = jnp.sum(m * w, axis=1, keepdims=True)
  c3 = jnp.sum(b3_ref[...] * wp_ref[...])
  pooled = sums / jnp.maximum(counts, 1.0)
  out_ref[...] = pooled + jnp.where(counts > 0.0, c3, 0.0) + bp_ref[...]


def _pool_call(accv, vp_row, dinv_row, batch_row, b3r, wpr, bpr):
  return pl.pallas_call(
      _pool_body,
      out_shape=jax.ShapeDtypeStruct((G, 1), jnp.float32),
  )(accv, vp_row, dinv_row, batch_row, b3r, wpr, bpr)


@jax.jit
def kernel(x, edge_index, batch, W1, b1, W2, b2, W3, b3, Wp, bp):
  src = edge_index[0]
  dst = edge_index[1]
  padn = EPAD - E
  srcp = jnp.concatenate([src, jnp.zeros((padn,), jnp.int32)])
  dstp = jnp.concatenate([dst, jnp.full((padn,), N, jnp.int32)])

  dst16 = dstp.reshape(NS, NCH, CK)
  src32 = srcp.reshape(32, NG32, 16)
  dst32 = dstp.reshape(32, NG32, 16)
  off2 = (srcp[None, :] + (jnp.arange(2, dtype=jnp.int32) * NPAD)[:, None])
  off4 = (srcp[None, :] + (jnp.arange(4, dtype=jnp.int32) * NPAD)[:, None])
  src2 = off2.reshape(2, NS, NCH, CK)
  src4 = off4.reshape(4, NS, NCH, CK)

  xpad = jnp.concatenate(
      [x.astype(jnp.float32), jnp.zeros((NPAD - N, F_IN), jnp.float32)])
  batchp = jnp.concatenate(
      [batch, jnp.full((NPAD - N,), G, jnp.int32)]).reshape(1, NPAD)
  zeros_blk = jnp.zeros((RZ, 128), jnp.float32)

  degp = _deg_kernel(dst32)
  dinv, xb = _prep_call(degp, xpad)
  accx = _spmm2(src2, dst16, xb.reshape(2 * NPAD, 128), zeros_blk)
  h1b = _layer1_call(accx.reshape(2, NPAD, 128), xb, dinv, W1,
                     b1.reshape(1, H))
  acch = _spmm4(src4, dst16, h1b.reshape(4 * NPAD, 128), zeros_blk)
  u = _u_call(W3, Wp)
  vp = _layer2_call(acch.reshape(4, NPAD, 128), h1b, dinv, W2,
                    b2.reshape(1, H), u)
  accv = _spmv_kernel(src32, dst32, vp.reshape(NPAD))
  return _pool_call(accv, vp.reshape(1, NPAD), dinv.reshape(1, NPAD),
                    batchp, b3.reshape(1, H), Wp.reshape(1, H),
                    bp.reshape(1, 1))


# speedup vs baseline: 11.2929x; 1.0004x over previous
import functools

import jax
import jax.numpy as jnp
from jax import lax
from jax.experimental import pallas as pl
from jax.experimental.pallas import tpu as pltpu
from jax.experimental.pallas import tpu_sc as plsc

N = 10000
NPAD = 10240
E = 160000
EPAD = 163840
F_IN = 256
H = 512
G = 64

NC = 2
NS = 16
CK = 64
NCH = EPAD // (NS * CK)
NSC = NCH // 4
EPT32 = EPAD // 32
NG32 = EPT32 // 16

BN = 512
GT = NPAD // BN
RZ = NPAD // NS

_mesh = plsc.VectorSubcoreMesh(core_axis_name="c", subcore_axis_name="s")
_sc_params = pltpu.CompilerParams(needs_layout_passes=False)


@functools.partial(
    pl.kernel,
    out_type=jax.ShapeDtypeStruct((32, NPAD), jnp.float32),
    mesh=_mesh,
    compiler_params=_sc_params,
    scratch_types=[
        pltpu.VMEM((NG32, 16), jnp.int32),
        pltpu.VMEM((NPAD,), jnp.float32),
    ],
)
def _deg_kernel(dst_hbm, out_hbm, dstb, hist):
  wid = lax.axis_index("s") * NC + lax.axis_index("c")
  zeros16 = jnp.zeros((16,), jnp.float32)
  ones16 = jnp.ones((16,), jnp.float32)

  def zero_body(i, _):
    hist[pl.ds(i * 16, 16)] = zeros16
    return 0

  lax.fori_loop(0, NPAD // 16, zero_body, 0)
  pltpu.sync_copy(dst_hbm.at[wid], dstb)

  def edge_body(j, _):
    idx = dstb[j]
    plsc.addupdate_scatter(hist, [idx], ones16)
    return 0

  lax.fori_loop(0, NG32, edge_body, 0)
  pltpu.sync_copy(hist, out_hbm.at[wid])


def _make_spmm(nb):
  nbps = nb // 2

  @functools.partial(
      pl.kernel,
      out_type=jax.ShapeDtypeStruct((nb * NPAD, 128), jnp.float32),
      mesh=_mesh,
      compiler_params=_sc_params,
      scratch_types=[
          pltpu.VMEM((2, 4, CK), jnp.int32),
          pltpu.VMEM((2, 4, CK), jnp.int32),
          pltpu.VMEM((CK, 128), jnp.float32),
          pltpu.VMEM((CK, 128), jnp.float32),
          pltpu.VMEM((CK, 128), jnp.float32),
          pltpu.VMEM((CK, 128), jnp.float32),
          pltpu.VMEM_SHARED((NPAD, 128), jnp.float32),
          pltpu.SemaphoreType.DMA,
          pltpu.SemaphoreType.DMA,
          pltpu.SemaphoreType.DMA,
          pltpu.SemaphoreType.DMA,
          pltpu.SemaphoreType.DMA,
          pltpu.SemaphoreType.DMA,
          pltpu.SemaphoreType.DMA,
          pltpu.SemaphoreType.DMA,
      ],
  )
  def spmm(src_hbm, dst_hbm, h_hbm, zeros_hbm, out_hbm,
           sidx, didx, r0, r1, r2, r3, acc,
           m0, m1, m2, m3, t0, t1, t2, t3):
    cid = lax.axis_index("c")
    sid = lax.axis_index("s")
    rows = (r0, r1, r2, r3)
    sems = (m0, m1, m2, m3)
    ssems = (t0, t1, t2, t3)

    def gather(p, c, buf):
      return pltpu.async_copy(h_hbm.at[sidx.at[p, c]], rows[buf], sems[buf])

    def gwait(p, c, buf):
      pltpu.make_async_copy(h_hbm.at[sidx.at[p, c]], rows[buf],
                            sems[buf]).wait()

    def scatter(p, c, buf):
      pltpu.async_copy(rows[buf], acc.at[didx.at[p, c]], ssems[buf],
                       add=True)

    def swait(p, c, buf):
      pltpu.make_async_copy(rows[buf], acc.at[didx.at[p, c]],
                            ssems[buf]).wait()

    for bi in range(nbps):
      b = cid * nbps + bi
      pltpu.sync_copy(zeros_hbm, acc.at[pl.ds(sid * RZ, RZ)])
      plsc.subcore_barrier()

      pltpu.sync_copy(src_hbm.at[b, sid, pl.ds(0, 4)], sidx.at[0])
      pltpu.sync_copy(dst_hbm.at[sid, pl.ds(0, 4)], didx.at[0])
      gather(0, 0, 0)
      gather(0, 1, 1)

      def superchunk(j, _):
        p = lax.rem(j, 2)
        pn = lax.rem(j + 1, 2)
        not_last = j < NSC - 1

        @pl.when(not_last)
        def _():
          pltpu.sync_copy(src_hbm.at[b, sid, pl.ds(4 * (j + 1), 4)],
                          sidx.at[pn])
          pltpu.sync_copy(dst_hbm.at[sid, pl.ds(4 * (j + 1), 4)],
                          didx.at[pn])

        gwait(p, 0, 0)

        @pl.when(j > 0)
        def _():
          swait(pn, 2, 2)

        gather(p, 2, 2)
        scatter(p, 0, 0)
        gwait(p, 1, 1)

        @pl.when(j > 0)
        def _():
          swait(pn, 3, 3)

        gather(p, 3, 3)
        scatter(p, 1, 1)
        gwait(p, 2, 2)

        @pl.when(not_last)
        def _():
          swait(p, 0, 0)
          gather(pn, 0, 0)

        scatter(p, 2, 2)
        gwait(p, 3, 3)

        @pl.when(not_last)
        def _():
          swait(p, 1, 1)
          gather(pn, 1, 1)

        scatter(p, 3, 3)
        return 0

      lax.fori_loop(0, NSC, superchunk, 0)
      pl_last = (NSC - 1) % 2
      swait(pl_last, 0, 0)
      swait(pl_last, 1, 1)
      swait(pl_last, 2, 2)
      swait(pl_last, 3, 3)
      plsc.subcore_barrier()
      pltpu.sync_copy(acc.at[pl.ds(sid * RZ, RZ)],
                      out_hbm.at[pl.ds(b * NPAD + sid * RZ, RZ)])
      plsc.subcore_barrier()

  return spmm


_spmm2 = _make_spmm(2)
_spmm4 = _make_spmm(4)


@functools.partial(
    pl.kernel,
    out_type=jax.ShapeDtypeStruct((32, NPAD), jnp.float32),
    mesh=_mesh,
    compiler_params=_sc_params,
    scratch_types=[
        pltpu.VMEM((NG32, 16), jnp.int32),
        pltpu.VMEM((NG32, 16), jnp.int32),
        pltpu.VMEM((NPAD,), jnp.float32),
        pltpu.VMEM((NPAD,), jnp.float32),
    ],
)
def _spmv_kernel(src_hbm, dst_hbm, v_hbm, out_hbm, srcb, dstb, vv, hist):
  wid = lax.axis_index("s") * NC + lax.axis_index("c")
  zeros16 = jnp.zeros((16,), jnp.float32)

  def zero_body(i, _):
    hist[pl.ds(i * 16, 16)] = zeros16
    return 0

  lax.fori_loop(0, NPAD // 16, zero_body, 0)
  pltpu.sync_copy(v_hbm, vv)
  pltpu.sync_copy(src_hbm.at[wid], srcb)
  pltpu.sync_copy(dst_hbm.at[wid], dstb)

  def edge_body(j, _):
    s16 = srcb[j]
    d16 = dstb[j]
    vals = plsc.load_gather(vv, [s16])
    plsc.addupdate_scatter(hist, [d16], vals)
    return 0

  lax.fori_loop(0, NG32, edge_body, 0)
  pltpu.sync_copy(hist, out_hbm.at[wid])


def _prep_body(degp_ref, x_ref, dinv_ref, xb_ref):
  ones32 = jnp.ones((32, 1), jnp.float32)
  deg = lax.dot_general(degp_ref[...], ones32, (((0,), (0,)), ((), ())),
                        preferred_element_type=jnp.float32)
  dinv = lax.rsqrt(deg + 1.0)
  dinv_ref[...] = dinv
  x = x_ref[...]
  for b in range(2):
    xb_ref[b] = dinv * x[:, b * 128:(b + 1) * 128]


def _prep_call(degp, xpad):
  return pl.pallas_call(
      _prep_body,
      grid=(GT,),
      in_specs=[
          pl.BlockSpec((32, BN), lambda i: (0, i)),
          pl.BlockSpec((BN, F_IN), lambda i: (i, 0)),
      ],
      out_specs=[
          pl.BlockSpec((BN, 1), lambda i: (i, 0)),
          pl.BlockSpec((2, BN, 128), lambda i: (0, i, 0)),
      ],
      out_shape=[
          jax.ShapeDtypeStruct((NPAD, 1), jnp.float32),
          jax.ShapeDtypeStruct((2, NPAD, 128), jnp.float32),
      ],
  )(degp, xpad)


def _layer1_body(accx_ref, xb_ref, dinv_ref, w1_ref, b1_ref, h1b_ref):
  dinv = dinv_ref[...]
  sx = jnp.concatenate(
      [dinv * (accx_ref[b] + xb_ref[b]) for b in range(2)], axis=1)
  h = jnp.dot(sx, w1_ref[...], preferred_element_type=jnp.float32)
  h = jnp.maximum(h + b1_ref[...], 0.0)
  for b in range(4):
    h1b_ref[b] = dinv * h[:, b * 128:(b + 1) * 128]


def _layer1_call(accx, xb, dinv, W1, b1r):
  return pl.pallas_call(
      _layer1_body,
      grid=(GT,),
      in_specs=[
          pl.BlockSpec((2, BN, 128), lambda i: (0, i, 0)),
          pl.BlockSpec((2, BN, 128), lambda i: (0, i, 0)),
          pl.BlockSpec((BN, 1), lambda i: (i, 0)),
          pl.BlockSpec((F_IN, H), lambda i: (0, 0)),
          pl.BlockSpec((1, H), lambda i: (0, 0)),
      ],
      out_specs=pl.BlockSpec((4, BN, 128), lambda i: (0, i, 0)),
      out_shape=jax.ShapeDtypeStruct((4, NPAD, 128), jnp.float32),
  )(accx, xb, dinv, W1, b1r)


def _layer2_body(acch_ref, h1b_ref, dinv_ref, w2_ref, b2_ref, u_ref, vp_ref):
  dinv = dinv_ref[...]
  sh = jnp.concatenate(
      [dinv * (acch_ref[b] + h1b_ref[b]) for b in range(4)], axis=1)
  h2 = jnp.dot(sh, w2_ref[...], preferred_element_type=jnp.float32)
  h2 = jnp.maximum(h2 + b2_ref[...], 0.0)
  v = jnp.dot(h2, u_ref[...], preferred_element_type=jnp.float32)
  vp_ref[...] = dinv * v


def _layer2_call(acch, h1b, dinv, W2, b2r, u):
  return pl.pallas_call(
      _layer2_body,
      grid=(GT,),
      in_specs=[
          pl.BlockSpec((4, BN, 128), lambda i: (0, i, 0)),
          pl.BlockSpec((4, BN, 128), lambda i: (0, i, 0)),
          pl.BlockSpec((BN, 1), lambda i: (i, 0)),
          pl.BlockSpec((H, H), lambda i: (0, 0)),
          pl.BlockSpec((1, H), lambda i: (0, 0)),
          pl.BlockSpec((H, 1), lambda i: (0, 0)),
      ],
      out_specs=pl.BlockSpec((BN, 1), lambda i: (i, 0)),
      out_shape=jax.ShapeDtypeStruct((NPAD, 1), jnp.float32),
  )(acch, h1b, dinv, W2, b2r, u)


def _u_body(w3_ref, wp_ref, u_ref):
  u_ref[...] = jnp.dot(w3_ref[...], wp_ref[...],
                       preferred_element_type=jnp.float32)


def _u_call(W3, Wp):
  return pl.pallas_call(
      _u_body,
      out_shape=jax.ShapeDtypeStruct((H, 1), jnp.float32),
  )(W3, Wp)


def _pool_body(accv_ref, vp_ref, dinv_ref, batch_ref, b3_ref, wp_ref, bp_ref,
               out_ref):
  accsum = jnp.sum(accv_ref[...], axis=0, keepdims=True)
  w = dinv_ref[...] * (accsum + vp_ref[...])
  ids = lax.broadcasted_iota(jnp.int32, (G, NPAD), 0)
  m = (batch_ref[...] == ids).astype(jnp.float32)
  counts = jnp.sum(m, axis=1, keepdims=True)
  sums = jnp.sum(m * w, axis=1, keepdims=True)
  c3 = jnp.sum(b3_ref[...] * wp_ref[...])
  pooled = sums / jnp.maximum(counts, 1.0)
  out_ref[...] = pooled + jnp.where(counts > 0.0, c3, 0.0) + bp_ref[...]


def _pool_call(accv, vp_row, dinv_row, batch_row, b3r, wpr, bpr):
  return pl.pallas_call(
      _pool_body,
      out_shape=jax.ShapeDtypeStruct((G, 1), jnp.float32),
  )(accv, vp_row, dinv_row, batch_row, b3r, wpr, bpr)


@jax.jit
def kernel(x, edge_index, batch, W1, b1, W2, b2, W3, b3, Wp, bp):
  src = edge_index[0]
  dst = edge_index[1]
  padn = EPAD - E
  srcp = jnp.concatenate([src, jnp.zeros((padn,), jnp.int32)])
  dstp = jnp.concatenate([dst, jnp.full((padn,), N, jnp.int32)])

  dst16 = dstp.reshape(NS, NCH, CK)
  src32 = srcp.reshape(32, NG32, 16)
  dst32 = dstp.reshape(32, NG32, 16)
  off2 = (srcp[None, :] + (jnp.arange(2, dtype=jnp.int32) * NPAD)[:, None])
  off4 = (srcp[None, :] + (jnp.arange(4, dtype=jnp.int32) * NPAD)[:, None])
  src2 = off2.reshape(2, NS, NCH, CK)
  src4 = off4.reshape(4, NS, NCH, CK)

  xpad = jnp.concatenate(
      [x.astype(jnp.float32), jnp.zeros((NPAD - N, F_IN), jnp.float32)])
  batchp = jnp.concatenate(
      [batch, jnp.full((NPAD - N,), G, jnp.int32)]).reshape(1, NPAD)
  zeros_blk = jnp.zeros((RZ, 128), jnp.float32)

  degp = _deg_kernel(dst32)
  dinv, xb = _prep_call(degp, xpad)
  accx = _spmm2(src2, dst16, xb.reshape(2 * NPAD, 128), zeros_blk)
  h1b = _layer1_call(accx.reshape(2, NPAD, 128), xb, dinv, W1,
                     b1.reshape(1, H))
  acch = _spmm4(src4, dst16, h1b.reshape(4 * NPAD, 128), zeros_blk)
  u = _u_call(W3, Wp)
  vp = _layer2_call(acch.reshape(4, NPAD, 128), h1b, dinv, W2,
                    b2.reshape(1, H), u)
  accv = _spmv_kernel(src32, dst32, vp.reshape(NPAD))
  return _pool_call(accv, vp.reshape(1, NPAD), dinv.reshape(1, NPAD),
                    batchp, b3.reshape(1, H), Wp.reshape(1, H),
                    bp.reshape(1, 1))
